# baseline (speedup 1.0000x reference)
import jax
import jax.numpy as jnp
from jax.experimental import pallas as pl

_E = 8
_ROWS_PER_BLOCK = 64


def _rr_gate_kernel(f32_ref, bool_ref):
    r, e, c = f32_ref.shape
    i = pl.program_id(0)
    base = i * r
    f32_ref[...] = jnp.zeros((r, e, c), f32_ref.dtype)
    bool_ref[...] = jnp.zeros((r, e, c), bool_ref.dtype)
    w = 128
    start = (base // _E) // w * w
    row = base + jax.lax.broadcasted_iota(jnp.int32, (r, e, w), 0)
    exp = jax.lax.broadcasted_iota(jnp.int32, (r, e, w), 1)
    cap = start + jax.lax.broadcasted_iota(jnp.int32, (r, e, w), 2)
    mask = (exp == (row % _E)) & (cap == (row // _E))
    f32_ref[:, :, pl.ds(start, w)] = mask.astype(f32_ref.dtype)
    bool_ref[:, :, pl.ds(start, w)] = mask.astype(bool_ref.dtype)


def kernel(input):
    s = input.shape[0]
    capacity = 2 * s // _E
    r = _ROWS_PER_BLOCK
    blk = (r, _E, capacity)
    f32_out, bool_out = pl.pallas_call(
        _rr_gate_kernel,
        grid=(s // r,),
        out_specs=[
            pl.BlockSpec(blk, lambda i: (i, 0, 0)),
            pl.BlockSpec(blk, lambda i: (i, 0, 0)),
        ],
        out_shape=[
            jax.ShapeDtypeStruct((s, _E, capacity), input.dtype),
            jax.ShapeDtypeStruct((s, _E, capacity), jnp.uint8),
        ],
    )()
    return (0.0, f32_out, bool_out.astype(jnp.bool_))


# speedup vs baseline: 1.1275x; 1.1275x over previous
import jax
import jax.numpy as jnp
from jax.experimental import pallas as pl

_E = 8
_ROWS_PER_BLOCK = 128


def _rr_gate_kernel(f32_ref, bool_ref):
    r, e, c = f32_ref.shape
    i = pl.program_id(0)
    base = i * r
    f32_ref[...] = jnp.zeros((r, e, c), f32_ref.dtype)
    bool_ref[...] = jnp.zeros((r, e, c), bool_ref.dtype)
    w = 128
    start = (base // _E) // w * w
    row = base + jax.lax.broadcasted_iota(jnp.int32, (r, e, w), 0)
    exp = jax.lax.broadcasted_iota(jnp.int32, (r, e, w), 1)
    cap = start + jax.lax.broadcasted_iota(jnp.int32, (r, e, w), 2)
    mask = (exp == (row % _E)) & (cap == (row // _E))
    f32_ref[:, :, pl.ds(start, w)] = mask.astype(f32_ref.dtype)
    bool_ref[:, :, pl.ds(start, w)] = mask.astype(bool_ref.dtype)


def kernel(input):
    s = input.shape[0]
    capacity = 2 * s // _E
    r = _ROWS_PER_BLOCK
    blk = (r, _E, capacity)
    f32_out, bool_out = pl.pallas_call(
        _rr_gate_kernel,
        grid=(s // r,),
        out_specs=[
            pl.BlockSpec(blk, lambda i: (i, 0, 0)),
            pl.BlockSpec(blk, lambda i: (i, 0, 0)),
        ],
        out_shape=[
            jax.ShapeDtypeStruct((s, _E, capacity), input.dtype),
            jax.ShapeDtypeStruct((s, _E, capacity), jnp.uint8),
        ],
    )()
    return (0.0, f32_out, bool_out.astype(jnp.bool_))
